# baseline (speedup 1.0000x reference)
import functools

import jax
import jax.numpy as jnp
from jax import lax
from jax.experimental import pallas as pl
from jax.experimental.pallas import tpu as pltpu
from jax.experimental.pallas import tpu_sc as plsc

NC = 2
NS = 16
IMIN = 128
CHUNK = 128
NBUF = 4
GPRE = 3


def _sc_scatter_add(src3, dst3, xt, cpw, acc_rows, n):
    dh = xt.shape[1]
    nvec = dh // 16
    zrows = acc_rows // NS
    wchunk = 80
    nwchunks = n // wchunk
    mesh = plsc.VectorSubcoreMesh(
        core_axis_name="c", subcore_axis_name="s", num_cores=NC, num_subcores=NS
    )

    @functools.partial(
        pl.kernel,
        mesh=mesh,
        compiler_params=pltpu.CompilerParams(use_tc_tiling_on_sc=False),
        out_type=jax.ShapeDtypeStruct((n, NC, dh), jnp.float32),
        scratch_types=[
            pltpu.VMEM((cpw, CHUNK), jnp.int32),
            pltpu.VMEM((cpw, CHUNK), jnp.int32),
            [pltpu.VMEM((CHUNK, dh), jnp.float32)] * NBUF,
            pltpu.VMEM_SHARED((acc_rows, dh), jnp.float32),
            [pltpu.SemaphoreType.DMA] * NBUF,
            [pltpu.SemaphoreType.DMA] * NBUF,
        ],
    )
    def k(src_hbm, dst_hbm, x_hbm, out_hbm, src_all, dst_all, rows, acc, gsem, ssem):
        cid = lax.axis_index("c")
        sid = lax.axis_index("s")

        pltpu.sync_copy(src_hbm.at[sid], src_all)
        pltpu.sync_copy(dst_hbm.at[sid], dst_all)

        def adjust(r, carry):
            for c in range(CHUNK // 16):
                sl = (r, pl.ds(c * 16, 16))
                src_all[sl] = src_all[sl] * 2 + cid
            return carry

        lax.fori_loop(0, cpw, adjust, 0)

        def zfill(r, carry):
            for c in range(nvec):
                rows[0][r, pl.ds(c * 16, 16)] = jnp.zeros((16,), jnp.float32)
            return carry

        lax.fori_loop(0, IMIN, zfill, 0)
        for b in range(zrows // IMIN):
            pltpu.sync_copy(
                rows[0].at[pl.ds(0, IMIN)],
                acc.at[pl.ds(sid * zrows + b * IMIN, IMIN)],
            )
        plsc.subcore_barrier()

        def start_gather(j, b):
            pltpu.async_copy(x_hbm.at[src_all.at[j]], rows[b], gsem[b])

        def drain(sem, b):
            pltpu.make_async_copy(x_hbm.at[pl.ds(0, CHUNK)], rows[b], sem).wait()

        for b in range(GPRE):
            start_gather(b, b)

        def gbody(g, carry):
            for b in range(NBUF):
                j = g * NBUF + b
                drain(gsem[b], b)
                pltpu.async_copy(rows[b], acc.at[dst_all.at[j]], ssem[b], add=True)
                jn = j + GPRE
                bn = (b + GPRE) % NBUF

                @pl.when(jn < cpw)
                def _():
                    @pl.when(jn >= NBUF)
                    def _():
                        drain(ssem[bn], bn)

                    start_gather(jn, bn)
            return carry

        lax.fori_loop(0, cpw // NBUF, gbody, 0)
        for b in range(NBUF):
            drain(ssem[b], b)
        plsc.subcore_barrier()

        def wbody(b, carry):
            t = b * NS + sid

            @pl.when(t < nwchunks)
            def _():
                r0 = t * wchunk
                pltpu.sync_copy(
                    acc.at[pl.ds(r0, wchunk)], rows[0].at[pl.ds(0, wchunk)]
                )
                pltpu.sync_copy(
                    rows[0].at[pl.ds(0, wchunk)], out_hbm.at[pl.ds(r0, wchunk), cid]
                )

            return carry

        lax.fori_loop(0, -(-nwchunks // NS), wbody, 0)

    return k(src3, dst3, xt)


def kernel(edge_index, x):
    n, d = x.shape
    dh = d // 2
    e = edge_index.shape[1]
    src = edge_index[0].astype(jnp.int32)
    dst = edge_index[1].astype(jnp.int32)

    cpw = -(-e // (NS * CHUNK))
    cpw = -(-cpw // NBUF) * NBUF
    e_pad = cpw * CHUNK * NS
    acc_rows = -(-(n + 1) // (NS * IMIN)) * (NS * IMIN)

    pad = e_pad - e
    if pad:
        src = jnp.concatenate([src, jnp.zeros((pad,), jnp.int32)])
        dst = jnp.concatenate([dst, jnp.full((pad,), n, jnp.int32)])

    src3 = src.reshape(NS, cpw, CHUNK)
    dst3 = dst.reshape(NS, cpw, CHUNK)
    xt = x.reshape(2 * n, dh)
    out3 = _sc_scatter_add(src3, dst3, xt, cpw, acc_rows, n)
    return out3.reshape(n, d)


# speedup vs baseline: 1.1721x; 1.1721x over previous
import functools

import jax
import jax.numpy as jnp
from jax import lax
from jax.experimental import pallas as pl
from jax.experimental.pallas import tpu as pltpu
from jax.experimental.pallas import tpu_sc as plsc

NC = 2
NS = 16
CHUNK = 128
NBUF = 4


def _sc_scatter_add(src4, dst3, xt, cpw, acc_rows, n):
    dh = xt.shape[1]
    nvec = dh // 16
    zrows = acc_rows // NS
    wchunk = 80
    nwchunks = n // wchunk
    mesh = plsc.VectorSubcoreMesh(
        core_axis_name="c", subcore_axis_name="s", num_cores=NC, num_subcores=NS
    )

    @functools.partial(
        pl.kernel,
        mesh=mesh,
        compiler_params=pltpu.CompilerParams(use_tc_tiling_on_sc=False),
        out_type=jax.ShapeDtypeStruct((n, NC, dh), jnp.float32),
        scratch_types=[
            pltpu.VMEM((cpw, CHUNK), jnp.int32),
            pltpu.VMEM((cpw, CHUNK), jnp.int32),
            [pltpu.VMEM((CHUNK, dh), jnp.float32)] * NBUF,
            pltpu.VMEM_SHARED((acc_rows, dh), jnp.float32),
            [pltpu.SemaphoreType.DMA] * NBUF,
            [pltpu.SemaphoreType.DMA] * NBUF,
        ],
    )
    def k(src_hbm, dst_hbm, x_hbm, out_hbm, src_all, dst_all, rows, acc, gsem, ssem):
        cid = lax.axis_index("c")
        sid = lax.axis_index("s")

        pltpu.sync_copy(src_hbm.at[cid, sid], src_all)
        pltpu.sync_copy(dst_hbm.at[sid], dst_all)

        def zfill(r, carry):
            for c in range(nvec):
                rows[0][r, pl.ds(c * 16, 16)] = jnp.zeros((16,), jnp.float32)
            return carry

        lax.fori_loop(0, CHUNK, zfill, 0)
        for b in range(zrows // CHUNK):
            pltpu.sync_copy(rows[0], acc.at[pl.ds(sid * zrows + b * CHUNK, CHUNK)])
        plsc.subcore_barrier()

        def start_gather(j, b):
            pltpu.async_copy(x_hbm.at[src_all.at[j]], rows[b], gsem[b])

        def drain(sem, b):
            pltpu.make_async_copy(x_hbm.at[pl.ds(0, CHUNK)], rows[b], sem).wait()

        for b in range(NBUF - 1):
            start_gather(b, b)

        def gbody(g, carry):
            for b in range(NBUF):
                j = g * NBUF + b
                drain(gsem[b], b)
                pltpu.async_copy(rows[b], acc.at[dst_all.at[j]], ssem[b], add=True)
                jn = j + NBUF - 1
                bn = (b + NBUF - 1) % NBUF

                @pl.when(jn < cpw)
                def _():
                    @pl.when(jn >= NBUF)
                    def _():
                        drain(ssem[bn], bn)

                    start_gather(jn, bn)
            return carry

        lax.fori_loop(0, cpw // NBUF, gbody, 0)
        for b in range(NBUF):
            drain(ssem[b], b)
        plsc.subcore_barrier()

        def wbody(b, carry):
            t = b * NS + sid

            @pl.when(t < nwchunks)
            def _():
                r0 = t * wchunk
                pltpu.sync_copy(
                    acc.at[pl.ds(r0, wchunk)], rows[0].at[pl.ds(0, wchunk)]
                )
                pltpu.sync_copy(
                    rows[0].at[pl.ds(0, wchunk)], out_hbm.at[pl.ds(r0, wchunk), cid]
                )

            return carry

        lax.fori_loop(0, -(-nwchunks // NS), wbody, 0)

    return k(src4, dst3, xt)


def kernel(edge_index, x):
    n, d = x.shape
    dh = d // 2
    e = edge_index.shape[1]
    src = edge_index[0].astype(jnp.int32)
    dst = edge_index[1].astype(jnp.int32)

    cpw = -(-e // (NS * CHUNK))
    cpw = -(-cpw // NBUF) * NBUF
    e_pad = cpw * CHUNK * NS
    acc_rows = -(-(n + 1) // (NS * CHUNK)) * (NS * CHUNK)

    pad = e_pad - e
    if pad:
        src = jnp.concatenate([src, jnp.zeros((pad,), jnp.int32)])
        dst = jnp.concatenate([dst, jnp.full((pad,), n, jnp.int32)])

    src3 = src.reshape(NS, cpw, CHUNK)
    src4 = jnp.stack([src3, src3 + n])
    dst3 = dst.reshape(NS, cpw, CHUNK)
    xt = x.reshape(n, 2, dh).transpose(1, 0, 2).reshape(2 * n, dh)
    out3 = _sc_scatter_add(src4, dst3, xt, cpw, acc_rows, n)
    return out3.reshape(n, d)


# speedup vs baseline: 1.2388x; 1.0569x over previous
import functools

import jax
import jax.numpy as jnp
from jax import lax
from jax.experimental import pallas as pl
from jax.experimental.pallas import tpu as pltpu
from jax.experimental.pallas import tpu_sc as plsc

NC = 2
NS = 16
CHUNK = 128
NBUF = 4


def _sc_scatter_add(src4, dst3, xt, cpw, acc_rows, n):
    dh = xt.shape[1]
    nvec = dh // 16
    zrows = acc_rows // NS
    wchunk = 80
    nwchunks = n // wchunk
    mesh = plsc.VectorSubcoreMesh(
        core_axis_name="c", subcore_axis_name="s", num_cores=NC, num_subcores=NS
    )

    @functools.partial(
        pl.kernel,
        mesh=mesh,
        compiler_params=pltpu.CompilerParams(use_tc_tiling_on_sc=False),
        out_type=jax.ShapeDtypeStruct((NC, n, dh), jnp.float32),
        scratch_types=[
            pltpu.VMEM((cpw, CHUNK), jnp.int32),
            pltpu.VMEM((cpw, CHUNK), jnp.int32),
            [pltpu.VMEM((CHUNK, dh), jnp.float32)] * NBUF,
            pltpu.VMEM_SHARED((acc_rows, dh), jnp.float32),
            [pltpu.SemaphoreType.DMA] * NBUF,
            [pltpu.SemaphoreType.DMA] * NBUF,
        ],
    )
    def k(src_hbm, dst_hbm, x_hbm, out_hbm, src_all, dst_all, rows, acc, gsem, ssem):
        cid = lax.axis_index("c")
        sid = lax.axis_index("s")

        pltpu.sync_copy(src_hbm.at[cid, sid], src_all)
        pltpu.sync_copy(dst_hbm.at[sid], dst_all)

        def zfill(r, carry):
            for c in range(nvec):
                rows[0][r, pl.ds(c * 16, 16)] = jnp.zeros((16,), jnp.float32)
            return carry

        lax.fori_loop(0, CHUNK, zfill, 0)
        for b in range(zrows // CHUNK):
            pltpu.sync_copy(rows[0], acc.at[pl.ds(sid * zrows + b * CHUNK, CHUNK)])
        plsc.subcore_barrier()

        def start_gather(j, b):
            pltpu.async_copy(x_hbm.at[src_all.at[j]], rows[b], gsem[b])

        def drain(sem, b):
            pltpu.make_async_copy(x_hbm.at[pl.ds(0, CHUNK)], rows[b], sem).wait()

        for b in range(NBUF - 1):
            start_gather(b, b)

        def gbody(g, carry):
            for b in range(NBUF):
                j = g * NBUF + b
                drain(gsem[b], b)
                pltpu.async_copy(rows[b], acc.at[dst_all.at[j]], ssem[b], add=True)
                jn = j + NBUF - 1
                bn = (b + NBUF - 1) % NBUF

                @pl.when(jn < cpw)
                def _():
                    @pl.when(jn >= NBUF)
                    def _():
                        drain(ssem[bn], bn)

                    start_gather(jn, bn)
            return carry

        lax.fori_loop(0, cpw // NBUF, gbody, 0)
        for b in range(NBUF):
            drain(ssem[b], b)
        plsc.subcore_barrier()

        def wbody(b, carry):
            t = b * NS + sid

            @pl.when(t < nwchunks)
            def _():
                r0 = t * wchunk
                pltpu.sync_copy(
                    acc.at[pl.ds(r0, wchunk)], rows[0].at[pl.ds(0, wchunk)]
                )
                pltpu.sync_copy(
                    rows[0].at[pl.ds(0, wchunk)], out_hbm.at[cid, pl.ds(r0, wchunk)]
                )

            return carry

        lax.fori_loop(0, -(-nwchunks // NS), wbody, 0)

    return k(src4, dst3, xt)


def kernel(edge_index, x):
    n, d = x.shape
    dh = d // 2
    e = edge_index.shape[1]
    src = edge_index[0].astype(jnp.int32)
    dst = edge_index[1].astype(jnp.int32)

    cpw = -(-e // (NS * CHUNK))
    cpw = -(-cpw // NBUF) * NBUF
    e_pad = cpw * CHUNK * NS
    acc_rows = -(-(n + 1) // (NS * CHUNK)) * (NS * CHUNK)

    pad = e_pad - e
    if pad:
        src = jnp.concatenate([src, jnp.zeros((pad,), jnp.int32)])
        dst = jnp.concatenate([dst, jnp.full((pad,), n, jnp.int32)])

    src3 = src.reshape(NS, cpw, CHUNK)
    src4 = jnp.stack([src3, src3 + n])
    dst3 = dst.reshape(NS, cpw, CHUNK)
    xt = x.reshape(n, 2, dh).transpose(1, 0, 2).reshape(2 * n, dh)
    out3 = _sc_scatter_add(src4, dst3, xt, cpw, acc_rows, n)
    return out3.transpose(1, 0, 2).reshape(n, d)


# speedup vs baseline: 1.2409x; 1.0017x over previous
import functools

import jax
import jax.numpy as jnp
from jax import lax
from jax.experimental import pallas as pl
from jax.experimental.pallas import tpu as pltpu
from jax.experimental.pallas import tpu_sc as plsc

NC = 2
NS = 16
CHUNK = 128
NBUF = 5


def _sc_scatter_add(src4, dst3, xt, cpw, acc_rows, n):
    dh = xt.shape[1]
    nvec = dh // 16
    zrows = acc_rows // NS
    wchunk = 80
    nwchunks = n // wchunk
    mesh = plsc.VectorSubcoreMesh(
        core_axis_name="c", subcore_axis_name="s", num_cores=NC, num_subcores=NS
    )

    @functools.partial(
        pl.kernel,
        mesh=mesh,
        compiler_params=pltpu.CompilerParams(use_tc_tiling_on_sc=False),
        out_type=jax.ShapeDtypeStruct((NC, n, dh), jnp.float32),
        scratch_types=[
            pltpu.VMEM((cpw, CHUNK), jnp.int32),
            pltpu.VMEM((cpw, CHUNK), jnp.int32),
            [pltpu.VMEM((CHUNK, dh), jnp.float32)] * NBUF,
            pltpu.VMEM_SHARED((acc_rows, dh), jnp.float32),
            [pltpu.SemaphoreType.DMA] * NBUF,
            [pltpu.SemaphoreType.DMA] * NBUF,
        ],
    )
    def k(src_hbm, dst_hbm, x_hbm, out_hbm, src_all, dst_all, rows, acc, gsem, ssem):
        cid = lax.axis_index("c")
        sid = lax.axis_index("s")

        pltpu.sync_copy(src_hbm.at[cid, sid], src_all)
        pltpu.sync_copy(dst_hbm.at[sid], dst_all)

        def zfill(r, carry):
            for c in range(nvec):
                rows[0][r, pl.ds(c * 16, 16)] = jnp.zeros((16,), jnp.float32)
            return carry

        lax.fori_loop(0, CHUNK, zfill, 0)
        for b in range(zrows // CHUNK):
            pltpu.sync_copy(rows[0], acc.at[pl.ds(sid * zrows + b * CHUNK, CHUNK)])
        plsc.subcore_barrier()

        def start_gather(j, b):
            pltpu.async_copy(x_hbm.at[src_all.at[j]], rows[b], gsem[b])

        def drain(sem, b):
            pltpu.make_async_copy(x_hbm.at[pl.ds(0, CHUNK)], rows[b], sem).wait()

        for b in range(3):
            start_gather(b, b)

        def gbody(g, carry):
            for b in range(NBUF):
                j = g * NBUF + b
                drain(gsem[b], b)
                pltpu.async_copy(rows[b], acc.at[dst_all.at[j]], ssem[b], add=True)
                jn = j + 3
                bn = (b + 3) % NBUF

                @pl.when(jn < cpw)
                def _():
                    @pl.when(jn >= NBUF)
                    def _():
                        drain(ssem[bn], bn)

                    start_gather(jn, bn)
            return carry

        lax.fori_loop(0, cpw // NBUF, gbody, 0)
        for b in range(NBUF):
            drain(ssem[b], b)
        plsc.subcore_barrier()

        def wbody(b, carry):
            t = b * NS + sid

            @pl.when(t < nwchunks)
            def _():
                r0 = t * wchunk
                pltpu.sync_copy(
                    acc.at[pl.ds(r0, wchunk)], rows[0].at[pl.ds(0, wchunk)]
                )
                pltpu.sync_copy(
                    rows[0].at[pl.ds(0, wchunk)], out_hbm.at[cid, pl.ds(r0, wchunk)]
                )

            return carry

        lax.fori_loop(0, -(-nwchunks // NS), wbody, 0)

    return k(src4, dst3, xt)


def kernel(edge_index, x):
    n, d = x.shape
    dh = d // 2
    e = edge_index.shape[1]
    src = edge_index[0].astype(jnp.int32)
    dst = edge_index[1].astype(jnp.int32)

    cpw = -(-e // (NS * CHUNK))
    cpw = -(-cpw // NBUF) * NBUF
    e_pad = cpw * CHUNK * NS
    acc_rows = -(-(n + 1) // (NS * CHUNK)) * (NS * CHUNK)

    pad = e_pad - e
    if pad:
        src = jnp.concatenate([src, jnp.zeros((pad,), jnp.int32)])
        dst = jnp.concatenate([dst, jnp.full((pad,), n, jnp.int32)])

    src3 = src.reshape(NS, cpw, CHUNK)
    src4 = jnp.stack([src3, src3 + n])
    dst3 = dst.reshape(NS, cpw, CHUNK)
    xt = x.reshape(n, 2, dh).transpose(1, 0, 2).reshape(2 * n, dh)
    out3 = _sc_scatter_add(src4, dst3, xt, cpw, acc_rows, n)
    return out3.transpose(1, 0, 2).reshape(n, d)
